# Initial kernel scaffold; baseline (speedup 1.0000x reference)
import functools

import jax
import jax.numpy as jnp
from jax import lax
from jax.experimental import pallas as pl
from jax.experimental.pallas import tpu as pltpu
from jax.experimental.pallas import tpu_sc as plsc

D = 64
B = 16384 * 50
NC, NS = 2, 16
NW = NC * NS
BPW = B // NW
CH = 128
NCHUNK = BPW // CH

_mesh = plsc.VectorSubcoreMesh(core_axis_name="c", subcore_axis_name="s")


@functools.partial(
    pl.kernel,
    out_type=jax.ShapeDtypeStruct((B, D), jnp.float32),
    mesh=_mesh,
    scratch_types=[
        pltpu.VMEM((NCHUNK, CH), jnp.int32),
        pltpu.VMEM((CH, D), jnp.float32),
        pltpu.SemaphoreType.DMA,
    ],
)
def _emb_lookup(idx_hbm, table_hbm, out_hbm, idx_v, rows_v, sem):
    wid = lax.axis_index("s") * NC + lax.axis_index("c")
    pltpu.sync_copy(idx_hbm.at[pl.ds(wid * NCHUNK, NCHUNK)], idx_v)
    out_base = wid * BPW

    @pl.loop(0, NCHUNK)
    def _chunk(j):
        pltpu.async_copy(table_hbm.at[idx_v.at[j]], rows_v, sem).wait()
        pltpu.sync_copy(rows_v, out_hbm.at[pl.ds(out_base + j * CH, CH)])


def kernel(input, weight):
    idx = input.reshape(NW * NCHUNK, CH).astype(jnp.int32)
    out = _emb_lookup(idx, weight)
    return out.reshape(input.shape[0], input.shape[1], D)


# speedup vs baseline: 1.6949x; 1.6949x over previous
import functools

import jax
import jax.numpy as jnp
from jax import lax
from jax.experimental import pallas as pl
from jax.experimental.pallas import tpu as pltpu
from jax.experimental.pallas import tpu_sc as plsc

D = 64
B = 16384 * 50
NC, NS = 2, 16
NW = NC * NS
BPW = B // NW
CH = 128
NCHUNK = BPW // CH

_mesh = plsc.VectorSubcoreMesh(core_axis_name="c", subcore_axis_name="s")


@functools.partial(
    pl.kernel,
    out_type=jax.ShapeDtypeStruct((B, D), jnp.float32),
    mesh=_mesh,
    scratch_types=[
        pltpu.VMEM((NCHUNK, CH), jnp.int32),
        pltpu.VMEM((CH, D), jnp.float32),
        pltpu.SemaphoreType.DMA,
    ],
    compiler_params=pltpu.CompilerParams(use_tc_tiling_on_sc=False),
)
def _emb_lookup(idx_hbm, table_hbm, out_hbm, idx_v, rows_v, sem):
    wid = lax.axis_index("s") * NC + lax.axis_index("c")
    pltpu.sync_copy(idx_hbm.at[pl.ds(wid * NCHUNK, NCHUNK)], idx_v)
    out_base = wid * BPW

    @pl.loop(0, NCHUNK)
    def _chunk(j):
        pltpu.async_copy(table_hbm.at[idx_v.at[j]], rows_v, sem).wait()
        pltpu.sync_copy(rows_v, out_hbm.at[pl.ds(out_base + j * CH, CH)])


def kernel(input, weight):
    idx = input.reshape(NW * NCHUNK, CH).astype(jnp.int32)
    out = _emb_lookup(idx, weight)
    return out.reshape(input.shape[0], input.shape[1], D)


# speedup vs baseline: 1.8739x; 1.1056x over previous
import functools

import jax
import jax.numpy as jnp
from jax import lax
from jax.experimental import pallas as pl
from jax.experimental.pallas import tpu as pltpu
from jax.experimental.pallas import tpu_sc as plsc

D = 64
B = 16384 * 50
NC, NS = 2, 16
NW = NC * NS
BPW = B // NW
CH = 128
NCHUNK = BPW // CH
NB = 4
NG = NCHUNK // NB

_mesh = plsc.VectorSubcoreMesh(core_axis_name="c", subcore_axis_name="s")


@functools.partial(
    pl.kernel,
    out_type=jax.ShapeDtypeStruct((B, D), jnp.float32),
    mesh=_mesh,
    scratch_types=[
        pltpu.VMEM((NCHUNK, CH), jnp.int32),
        pltpu.VMEM((2, NB, CH, D), jnp.float32),
        pltpu.SemaphoreType.DMA,
        pltpu.SemaphoreType.DMA,
        pltpu.SemaphoreType.DMA,
        pltpu.SemaphoreType.DMA,
    ],
    compiler_params=pltpu.CompilerParams(use_tc_tiling_on_sc=False),
)
def _emb_lookup(idx_hbm, table_hbm, out_hbm, idx_v, rows_v, sg0, sg1, sw0, sw1):
    wid = lax.axis_index("s") * NC + lax.axis_index("c")
    pltpu.sync_copy(idx_hbm.at[pl.ds(wid * NCHUNK, NCHUNK)], idx_v)
    out_base = wid * BPW
    sgs = (sg0, sg1)
    sws = (sw0, sw1)

    def fire_gathers(g, s):
        for b in range(NB):
            pltpu.async_copy(
                table_hbm.at[idx_v.at[g * NB + b]], rows_v.at[s, b], sgs[s])

    def wait_gathers(s):
        for b in range(NB):
            pltpu.make_async_copy(
                table_hbm.at[idx_v.at[b]], rows_v.at[s, b], sgs[s]).wait()

    def fire_writes(g, s):
        for b in range(NB):
            pltpu.async_copy(
                rows_v.at[s, b],
                out_hbm.at[pl.ds(out_base + (g * NB + b) * CH, CH)], sws[s])

    def wait_writes(s):
        for b in range(NB):
            pltpu.make_async_copy(
                rows_v.at[s, b],
                out_hbm.at[pl.ds(out_base + b * CH, CH)], sws[s]).wait()

    fire_gathers(0, 0)
    wait_gathers(0)
    fire_writes(0, 0)
    fire_gathers(1, 1)

    @pl.loop(0, (NG - 2) // 2)
    def _pair(t):
        g = 1 + 2 * t
        wait_writes(0)
        fire_gathers(g + 1, 0)
        wait_gathers(1)
        fire_writes(g, 1)
        wait_writes(1)
        fire_gathers(g + 2, 1)
        wait_gathers(0)
        fire_writes(g + 1, 0)

    wait_writes(0)
    wait_gathers(1)
    fire_writes(NG - 1, 1)
    wait_writes(1)


def kernel(input, weight):
    idx = input.reshape(NW * NCHUNK, CH).astype(jnp.int32)
    out = _emb_lookup(idx, weight)
    return out.reshape(input.shape[0], input.shape[1], D)
